# baseline (speedup 1.0000x reference)
import jax
import jax.numpy as jnp
from jax.experimental import pallas as pl

N = 32768
IN_DIM = 2048
HIDDEN = 32
TOPK = 3276

_BN = 512


def _scores_body(x_ref, w1_ref, b1_ref, w2_ref, b2_ref, out_ref):
    xb = x_ref[...]
    h = jnp.tanh(
        jnp.dot(xb, w1_ref[...], preferred_element_type=jnp.float32)
        + b1_ref[...]
    )
    s = jnp.dot(h, w2_ref[...], preferred_element_type=jnp.float32) + b2_ref[...]
    out_ref[...] = s


def _scores(x, W1, b1, W2, b2):
    grid = (N // _BN,)
    return pl.pallas_call(
        _scores_body,
        grid=grid,
        in_specs=[
            pl.BlockSpec((_BN, IN_DIM), lambda i: (i, 0)),
            pl.BlockSpec((IN_DIM, HIDDEN), lambda i: (0, 0)),
            pl.BlockSpec((1, HIDDEN), lambda i: (0, 0)),
            pl.BlockSpec((HIDDEN, 1), lambda i: (0, 0)),
            pl.BlockSpec((1, 1), lambda i: (0, 0)),
        ],
        out_specs=pl.BlockSpec((_BN, 1), lambda i: (i, 0)),
        out_shape=jax.ShapeDtypeStruct((N, 1), jnp.float32),
    )(x, W1, b1.reshape(1, HIDDEN), W2, b2.reshape(1, 1))


def kernel(x, W1, b1, W2, b2):
    s = _scores(x, W1, b1, W2, b2)
    A = s - jnp.max(s)
    A = jax.nn.softmax(A, axis=0)
    _, top_idx = jax.lax.top_k(jnp.squeeze(A, axis=-1), TOPK)
    x_sel = x[top_idx]
    A_sel = A[top_idx]
    A_sel = A_sel / jnp.sum(A_sel)
    M = jnp.sum(A_sel * x_sel, axis=0)
    return (M, A_sel)


# speedup vs baseline: 1.0059x; 1.0059x over previous
import dataclasses
import functools

import jax
import jax.numpy as jnp
from jax import lax
from jax.experimental import pallas as pl
from jax.experimental.pallas import tpu as pltpu
from jax.experimental.pallas import tpu_sc as plsc

N = 32768
IN_DIM = 2048
HIDDEN = 32
TOPK = 3276

_BN = 2048


def _scores_body(x_ref, w1_ref, b1_ref, w2_ref, b2_ref, out_ref):
    xb = x_ref[...]
    h = jnp.tanh(
        jnp.dot(xb, w1_ref[...], preferred_element_type=jnp.float32)
        + b1_ref[...]
    )
    s = jnp.dot(h, w2_ref[...], preferred_element_type=jnp.float32) + b2_ref[...]
    out_ref[...] = s


def _scores(x, W1, b1, W2, b2):
    grid = (N // _BN,)
    return pl.pallas_call(
        _scores_body,
        grid=grid,
        in_specs=[
            pl.BlockSpec((_BN, IN_DIM), lambda i: (i, 0)),
            pl.BlockSpec((IN_DIM, HIDDEN), lambda i: (0, 0)),
            pl.BlockSpec((1, HIDDEN), lambda i: (0, 0)),
            pl.BlockSpec((HIDDEN, 1), lambda i: (0, 0)),
            pl.BlockSpec((1, 1), lambda i: (0, 0)),
        ],
        out_specs=pl.BlockSpec((_BN, 1), lambda i: (i, 0)),
        out_shape=jax.ShapeDtypeStruct((N, 1), jnp.float32),
    )(x, W1, b1.reshape(1, HIDDEN), W2, b2.reshape(1, 1))


def _thresh_body(s_ref, meta_ref):
    sv = s_ref[...]
    m = jnp.max(sv)
    bits = jax.lax.bitcast_convert_type(sv, jnp.int32)
    su = bits ^ ((bits >> 31) & jnp.int32(0x7FFFFFFF))

    MIN32 = jnp.int32(-2147483648)

    def body(i, tu):
        candu = tu | (jnp.int32(1) << (jnp.int32(31) - i))
        cs = candu ^ MIN32
        cnt = jnp.sum((su >= cs).astype(jnp.int32))
        return jnp.where(cnt >= TOPK, candu, tu)

    T = jax.lax.fori_loop(0, 32, body, jnp.int32(0)) ^ MIN32
    n_gt = jnp.sum((su > T).astype(jnp.int32))
    n_tie = TOPK - n_gt
    ex = jnp.exp(sv - m)
    sT = jnp.where(T >= 0, T, T ^ jnp.int32(0x7FFFFFFF))
    eT = jnp.exp(jax.lax.bitcast_convert_type(sT, jnp.float32) - m)
    etot = (jnp.sum(jnp.where(su > T, ex, 0.0))
            + n_tie.astype(jnp.float32) * eT)
    meta_ref[0:1, :] = jnp.full((1, 128), T, jnp.int32)
    meta_ref[1:2, :] = jnp.full((1, 128), n_tie, jnp.int32)
    meta_ref[2:3, :] = jnp.full(
        (1, 128), jax.lax.bitcast_convert_type(m, jnp.int32), jnp.int32
    )
    meta_ref[3:4, :] = jnp.full(
        (1, 128), jax.lax.bitcast_convert_type(etot, jnp.int32), jnp.int32
    )
    meta_ref[4:8, :] = jnp.zeros((4, 128), jnp.int32)


def _thresh(s2d):
    return pl.pallas_call(
        _thresh_body,
        in_specs=[pl.BlockSpec((256, 128), lambda: (0, 0))],
        out_specs=pl.BlockSpec((8, 128), lambda: (0, 0)),
        out_shape=jax.ShapeDtypeStruct((8, 128), jnp.int32),
    )(s2d)


_NT = 16
_CH = N // _NT
_NV = _CH // 16
_GD = IN_DIM // 16


def _sc_body(s_hbm, meta_hbm, x_hbm, gid_hbm, asel_hbm, mpart_hbm,
             sv, valbuf, idxbuf, sval, meta_v, tstage, tblc_v,
             chunkbuf, rankall, wall, rankstage, wstage, gstage0,
             gstage1, wtmp, rows0, rows1, acc, tblc, tblv,
             sem0, sem1, sem2):
    c = lax.axis_index("c")
    t = lax.axis_index("s")
    base = t * _CH
    i32 = jnp.int32
    iota = lax.iota(i32, 16)
    zi = jnp.zeros((16,), i32)

    pltpu.sync_copy(s_hbm.at[pl.ds(base, _CH)], sv)
    pltpu.sync_copy(meta_hbm, meta_v)
    Tv = meta_v[pl.ds(0, 16)]
    ntie_v = meta_v[pl.ds(128, 16)]
    mval = plsc.bitcast(meta_v[pl.ds(256, 16)], jnp.float32)
    ntie = jnp.max(ntie_v)

    def _su(i):
        b = plsc.bitcast(sv[pl.ds(i * 16, 16)], i32)
        return b ^ ((b >> 31) & i32(0x7FFFFFFF))

    def p1(i, cnts):
        su = _su(i)
        return (cnts[0] + (su > Tv).astype(i32),
                cnts[1] + (su == Tv).astype(i32))

    cgt_v, ceq_v = lax.fori_loop(0, _NV, p1, (zi, zi))
    cnt_gt = jnp.sum(cgt_v)
    cnt_eq = jnp.sum(ceq_v)

    tstage[...] = jnp.where(iota == 0, cnt_gt,
                            jnp.where(iota == 1, cnt_eq, 0))
    pltpu.sync_copy(tstage, tblc.at[t])
    plsc.subcore_barrier()
    pltpu.sync_copy(tblc, tblc_v)
    gt_col = plsc.load_gather(tblc_v, [iota, zi])
    eq_col = plsc.load_gather(tblc_v, [iota, zi + 1])
    eq_pre = plsc.cumsum(eq_col) - eq_col
    tie_col = jnp.clip(ntie - eq_pre, 0, eq_col)
    sel_col = gt_col + tie_col
    my_tie = jnp.sum(tie_col * (iota == t).astype(i32))
    my_cnt = cnt_gt + my_tie

    def p2(i, carry):
        off, eqs = carry
        su = _su(i)
        s16 = sv[pl.ds(i * 16, 16)]
        gt = su > Tv
        eqi = (su == Tv).astype(i32)
        eq_ex = plsc.cumsum(eqi) - eqi
        sel = gt | ((eqi == 1) & ((eqs + eq_ex) < my_tie))
        seli = sel.astype(i32)
        pos = off + plsc.cumsum(seli) - seli
        plsc.store_scatter(valbuf, [pos], s16, mask=sel)
        plsc.store_scatter(idxbuf, [pos], base + i * 16 + iota, mask=sel)
        return (off + jnp.sum(seli), eqs + jnp.sum(eqi))

    lax.fori_loop(0, _NV, p2, (i32(0), i32(0)))
    nv = (my_cnt + 15) // 16
    plsc.store_scatter(valbuf, [my_cnt + iota],
                       jnp.full((16,), -jnp.inf, jnp.float32))

    def rank_one(e, _):
        ebc = jnp.full((16,), e, i32)
        ev = plsc.load_gather(valbuf, [ebc])

        def cb(v, a):
            vals = valbuf[pl.ds(v * 16, 16)]
            gpos = v * 16 + iota
            ahead = (vals > ev) | ((vals == ev) & (gpos < e))
            return a + ahead.astype(i32)

        r = jnp.sum(lax.fori_loop(0, nv, cb, zi))
        plsc.store_scatter(sval, [jnp.full((16,), r, i32)], ev,
                           mask=iota == 0)
        return _

    lax.fori_loop(0, my_cnt, rank_one, 0)

    pltpu.sync_copy(sval, tblv.at[t])
    plsc.subcore_barrier()

    def init_vreg(p, esum):
        pg = p * 16 + iota
        mv16 = plsc.load_gather(sval, [jnp.minimum(pg, my_cnt - 1)])
        valid = pg < my_cnt
        e16 = jnp.where(valid, jnp.exp(mv16 - mval), 0.0)
        rankall[pl.ds(p * 16, 16)] = pg
        wall[pl.ds(p * 16, 16)] = e16
        return esum + e16

    lax.fori_loop(0, nv, init_vreg, jnp.zeros((16,), jnp.float32))

    for u in range(_NT):
        pltpu.sync_copy(tblv.at[u], chunkbuf)
        cu = jnp.sum(sel_col * (iota == u).astype(i32))
        ge_u = (i32(u) < t).astype(i32)

        def bs_vreg(p, _, cu=cu, ge_u=ge_u, u=u):
            pg = p * 16 + iota
            mv16 = plsc.load_gather(sval, [jnp.minimum(pg, my_cnt - 1)])
            lo = zi
            hi = jnp.full((16,), cu, i32)
            for _step in range(11):
                mid = jnp.minimum((lo + hi) >> 1, i32(_CH - 1))
                av = plsc.load_gather(chunkbuf, [mid])
                pred = (av > mv16) | ((av == mv16) & (ge_u == 1))
                lo = jnp.where(pred, mid + 1, lo)
                hi = jnp.where(pred, hi, mid)
            cnt_u = jnp.minimum(lo, cu) * (i32(u) != t).astype(i32)
            rankall[pl.ds(p * 16, 16)] = rankall[pl.ds(p * 16, 16)] + cnt_u
            return _

        lax.fori_loop(0, nv, bs_vreg, 0)

    etot = plsc.bitcast(meta_v[pl.ds(384, 16)], jnp.float32)

    @pl.loop(0, _GD)
    def _zacc(v):
        acc[v] = jnp.zeros((16,), jnp.float32)

    @pl.when(c == 0)
    def _asel():
        def sc_one(p, _):
            pg = p * 16 + iota
            valid = pg < my_cnt
            pgc = jnp.minimum(pg, my_cnt - 1)
            r16 = plsc.load_gather(rankall, [pgc])
            w16 = plsc.load_gather(wall, [pgc]) / etot
            rankstage[...] = r16
            wstage[...] = w16
            pltpu.async_copy(wstage, asel_hbm.at[rankstage], sem2).wait()
            return _

        lax.fori_loop(0, nv, sc_one, 0)

    half = (my_cnt + 1) // 2
    start = c * half
    nc = jnp.where(c == 0, half, my_cnt - half)
    nch = (nc + 15) // 16

    def do_chunk(cc, gstage, rows, sem):
        o = start + cc * 16
        valid = (o + iota) < (start + nc)
        safe0 = plsc.load_gather(idxbuf, [jnp.full((16,), start, i32)])
        idx16 = plsc.load_gather(idxbuf, [jnp.minimum(o + iota,
                                                      my_cnt - 1)])
        gstage[...] = jnp.where(valid, idx16, safe0)
        return pltpu.async_copy(x_hbm.at[gstage], rows, sem)

    def acc_chunk(cc, rows):
        o = start + cc * 16
        valid = (o + iota) < (start + nc)
        v16 = plsc.load_gather(valbuf, [jnp.minimum(o + iota, my_cnt - 1)])
        w16 = jnp.where(valid, jnp.exp(v16 - mval), 0.0) / etot
        wb = [jnp.sum(jnp.where(iota == r, w16, 0.0)) for r in range(16)]

        @pl.loop(0, _GD)
        def _av(v):
            a = acc[v]
            for r in range(16):
                a = a + wb[r] * rows[r, pl.ds(v * 16, 16)]
            acc[v] = a

    @pl.loop(0, nch)
    def _chunk(cc):
        do_chunk(cc, gstage0, rows0, sem0).wait()
        acc_chunk(cc, rows0)

    pltpu.sync_copy(acc, mpart_hbm.at[c * _NT + t])


def _sc_select(s1d, meta1d, x):
    mesh = plsc.VectorSubcoreMesh(core_axis_name="c", subcore_axis_name="s")
    cp = pltpu.CompilerParams()
    if "needs_layout_passes" in pltpu.CompilerParams.__dataclass_fields__:
        cp = dataclasses.replace(cp, needs_layout_passes=False)
    f32 = jnp.float32
    i32 = jnp.int32
    kern = functools.partial(
        pl.kernel,
        out_type=(jax.ShapeDtypeStruct((TOPK,), f32),
                  jax.ShapeDtypeStruct((2 * _NT, _GD, 16), f32)),
        mesh=mesh,
        compiler_params=cp,
        scratch_types=[
            pltpu.VMEM((_CH,), f32),
            pltpu.VMEM((_CH + 16,), f32),
            pltpu.VMEM((_CH + 16,), i32),
            pltpu.VMEM((_CH,), f32),
            pltpu.VMEM((1024,), i32),
            pltpu.VMEM((16,), i32),
            pltpu.VMEM((16, 16), i32),
            pltpu.VMEM((_CH,), f32),
            pltpu.VMEM((_CH,), i32),
            pltpu.VMEM((_CH,), f32),
            pltpu.VMEM((16,), i32),
            pltpu.VMEM((16,), f32),
            pltpu.VMEM((16,), i32),
            pltpu.VMEM((16,), i32),
            pltpu.VMEM((16,), f32),
            pltpu.VMEM((16, IN_DIM), f32),
            pltpu.VMEM((16, IN_DIM), f32),
            pltpu.VMEM((_GD, 16), f32),
            pltpu.VMEM_SHARED((_NT, 16), i32),
            pltpu.VMEM_SHARED((_NT, _CH), f32),
            pltpu.SemaphoreType.DMA,
            pltpu.SemaphoreType.DMA,
            pltpu.SemaphoreType.DMA,
        ],
    )(_sc_body)
    return kern(s1d, meta1d, x, jnp.arange(_GD, dtype=i32))


def kernel(x, W1, b1, W2, b2):
    s = _scores(x, W1, b1, W2, b2)
    meta = _thresh(s.reshape(256, 128))
    asel, mpart = _sc_select(s.reshape(N), meta.reshape(1024), x)
    return (jnp.sum(mpart.reshape(2 * _NT, IN_DIM), axis=0), asel[:, None])


# speedup vs baseline: 1.0153x; 1.0094x over previous
import dataclasses
import functools

import jax
import jax.numpy as jnp
from jax import lax
from jax.experimental import pallas as pl
from jax.experimental.pallas import tpu as pltpu
from jax.experimental.pallas import tpu_sc as plsc

N = 32768
IN_DIM = 2048
HIDDEN = 32
TOPK = 3276

_BN = 2048


def _scores_body(x_ref, w1_ref, b1_ref, w2_ref, b2_ref, out_ref):
    xb = x_ref[...]
    h = jnp.tanh(
        jnp.dot(xb, w1_ref[...], preferred_element_type=jnp.float32)
        + b1_ref[...]
    )
    s = jnp.dot(h, w2_ref[...], preferred_element_type=jnp.float32) + b2_ref[...]
    out_ref[...] = s


def _scores(x, W1, b1, W2, b2):
    grid = (N // _BN,)
    return pl.pallas_call(
        _scores_body,
        grid=grid,
        in_specs=[
            pl.BlockSpec((_BN, IN_DIM), lambda i: (i, 0)),
            pl.BlockSpec((IN_DIM, HIDDEN), lambda i: (0, 0)),
            pl.BlockSpec((1, HIDDEN), lambda i: (0, 0)),
            pl.BlockSpec((HIDDEN, 1), lambda i: (0, 0)),
            pl.BlockSpec((1, 1), lambda i: (0, 0)),
        ],
        out_specs=pl.BlockSpec((_BN, 1), lambda i: (i, 0)),
        out_shape=jax.ShapeDtypeStruct((N, 1), jnp.float32),
    )(x, W1, b1.reshape(1, HIDDEN), W2, b2.reshape(1, 1))


def _thresh_body(s_ref, meta_ref):
    sv = s_ref[...]
    m = jnp.max(sv)
    bits = jax.lax.bitcast_convert_type(sv, jnp.int32)
    su = bits ^ ((bits >> 31) & jnp.int32(0x7FFFFFFF))

    MIN32 = jnp.int32(-2147483648)

    def body(i, tu):
        candu = tu | (jnp.int32(1) << (jnp.int32(31) - i))
        cs = candu ^ MIN32
        cnt = jnp.sum((su >= cs).astype(jnp.int32))
        return jnp.where(cnt >= TOPK, candu, tu)

    T = jax.lax.fori_loop(0, 32, body, jnp.int32(0)) ^ MIN32
    n_gt = jnp.sum((su > T).astype(jnp.int32))
    n_tie = TOPK - n_gt
    ex = jnp.exp(sv - m)
    sT = jnp.where(T >= 0, T, T ^ jnp.int32(0x7FFFFFFF))
    eT = jnp.exp(jax.lax.bitcast_convert_type(sT, jnp.float32) - m)
    etot = (jnp.sum(jnp.where(su > T, ex, 0.0))
            + n_tie.astype(jnp.float32) * eT)
    meta_ref[0:1, :] = jnp.full((1, 128), T, jnp.int32)
    meta_ref[1:2, :] = jnp.full((1, 128), n_tie, jnp.int32)
    meta_ref[2:3, :] = jnp.full(
        (1, 128), jax.lax.bitcast_convert_type(m, jnp.int32), jnp.int32
    )
    meta_ref[3:4, :] = jnp.full(
        (1, 128), jax.lax.bitcast_convert_type(etot, jnp.int32), jnp.int32
    )
    meta_ref[4:8, :] = jnp.zeros((4, 128), jnp.int32)


def _thresh(s2d):
    return pl.pallas_call(
        _thresh_body,
        in_specs=[pl.BlockSpec((256, 128), lambda: (0, 0))],
        out_specs=pl.BlockSpec((8, 128), lambda: (0, 0)),
        out_shape=jax.ShapeDtypeStruct((8, 128), jnp.int32),
    )(s2d)


_NT = 16
_CH = N // _NT
_NV = _CH // 16
_GD = IN_DIM // 16


def _sc_body(s_hbm, meta_hbm, x_hbm, gid_hbm, asel_hbm, mpart_hbm,
             sv, valbuf, idxbuf, sval, meta_v, tstage, tblc_v,
             chunkbuf, rankall, wall, rankstage, wstage, gstage0,
             gstage1, wtmp, rows0, rows1, acc, tblc, tblv,
             sem0, sem1, sem2):
    c = lax.axis_index("c")
    t = lax.axis_index("s")
    base = t * _CH
    i32 = jnp.int32
    iota = lax.iota(i32, 16)
    zi = jnp.zeros((16,), i32)

    pltpu.sync_copy(s_hbm.at[pl.ds(base, _CH)], sv)
    pltpu.sync_copy(meta_hbm, meta_v)
    Tv = meta_v[pl.ds(0, 16)]
    ntie_v = meta_v[pl.ds(128, 16)]
    mval = plsc.bitcast(meta_v[pl.ds(256, 16)], jnp.float32)
    ntie = jnp.max(ntie_v)

    def _su(i):
        b = plsc.bitcast(sv[pl.ds(i * 16, 16)], i32)
        return b ^ ((b >> 31) & i32(0x7FFFFFFF))

    def p1(i, cnts):
        su = _su(i)
        return (cnts[0] + (su > Tv).astype(i32),
                cnts[1] + (su == Tv).astype(i32))

    cgt_v, ceq_v = lax.fori_loop(0, _NV, p1, (zi, zi))
    cnt_gt = jnp.sum(cgt_v)
    cnt_eq = jnp.sum(ceq_v)

    tstage[...] = jnp.where(iota == 0, cnt_gt,
                            jnp.where(iota == 1, cnt_eq, 0))
    pltpu.sync_copy(tstage, tblc.at[t])
    plsc.subcore_barrier()
    pltpu.sync_copy(tblc, tblc_v)
    gt_col = plsc.load_gather(tblc_v, [iota, zi])
    eq_col = plsc.load_gather(tblc_v, [iota, zi + 1])
    eq_pre = plsc.cumsum(eq_col) - eq_col
    tie_col = jnp.clip(ntie - eq_pre, 0, eq_col)
    sel_col = gt_col + tie_col
    my_tie = jnp.sum(tie_col * (iota == t).astype(i32))
    my_cnt = cnt_gt + my_tie

    def p2(i, carry):
        off, eqs = carry
        su = _su(i)
        s16 = sv[pl.ds(i * 16, 16)]
        gt = su > Tv
        eqi = (su == Tv).astype(i32)
        eq_ex = plsc.cumsum(eqi) - eqi
        sel = gt | ((eqi == 1) & ((eqs + eq_ex) < my_tie))
        seli = sel.astype(i32)
        pos = off + plsc.cumsum(seli) - seli
        plsc.store_scatter(valbuf, [pos], s16, mask=sel)
        plsc.store_scatter(idxbuf, [pos], base + i * 16 + iota, mask=sel)
        return (off + jnp.sum(seli), eqs + jnp.sum(eqi))

    lax.fori_loop(0, _NV, p2, (i32(0), i32(0)))
    nv = (my_cnt + 15) // 16
    plsc.store_scatter(valbuf, [my_cnt + iota],
                       jnp.full((16,), -jnp.inf, jnp.float32))

    def rank_one(e, _):
        ebc = jnp.full((16,), e, i32)
        ev = plsc.load_gather(valbuf, [ebc])

        def cb(v, a):
            vals = valbuf[pl.ds(v * 16, 16)]
            gpos = v * 16 + iota
            ahead = (vals > ev) | ((vals == ev) & (gpos < e))
            return a + ahead.astype(i32)

        r = jnp.sum(lax.fori_loop(0, nv, cb, zi))
        plsc.store_scatter(sval, [jnp.full((16,), r, i32)], ev,
                           mask=iota == 0)
        return _

    lax.fori_loop(0, my_cnt, rank_one, 0)

    pltpu.sync_copy(sval, tblv.at[t])
    plsc.subcore_barrier()

    def init_vreg(p, esum):
        pg = p * 16 + iota
        mv16 = plsc.load_gather(sval, [jnp.minimum(pg, my_cnt - 1)])
        valid = pg < my_cnt
        e16 = jnp.where(valid, jnp.exp(mv16 - mval), 0.0)
        rankall[pl.ds(p * 16, 16)] = pg
        wall[pl.ds(p * 16, 16)] = e16
        return esum + e16

    lax.fori_loop(0, nv, init_vreg, jnp.zeros((16,), jnp.float32))

    for u in range(_NT):
        pltpu.sync_copy(tblv.at[u], chunkbuf)
        cu = jnp.sum(sel_col * (iota == u).astype(i32))
        ge_u = (i32(u) < t).astype(i32)

        def bs_vreg(p, _, cu=cu, ge_u=ge_u, u=u):
            pg = p * 16 + iota
            mv16 = plsc.load_gather(sval, [jnp.minimum(pg, my_cnt - 1)])
            lo = zi
            hi = jnp.full((16,), cu, i32)
            for _step in range(11):
                mid = jnp.minimum((lo + hi) >> 1, i32(_CH - 1))
                av = plsc.load_gather(chunkbuf, [mid])
                pred = (av > mv16) | ((av == mv16) & (ge_u == 1))
                lo = jnp.where(pred, mid + 1, lo)
                hi = jnp.where(pred, hi, mid)
            cnt_u = jnp.minimum(lo, cu) * (i32(u) != t).astype(i32)
            rankall[pl.ds(p * 16, 16)] = rankall[pl.ds(p * 16, 16)] + cnt_u
            return _

        lax.fori_loop(0, nv, bs_vreg, 0)

    etot = plsc.bitcast(meta_v[pl.ds(384, 16)], jnp.float32)

    @pl.loop(0, _GD)
    def _zacc(v):
        acc[v] = jnp.zeros((16,), jnp.float32)

    @pl.when(c == 0)
    def _asel():
        def sc_one(p, _):
            pg = p * 16 + iota
            valid = pg < my_cnt
            pgc = jnp.minimum(pg, my_cnt - 1)
            r16 = plsc.load_gather(rankall, [pgc])
            w16 = plsc.load_gather(wall, [pgc]) / etot
            rankstage[...] = r16
            wstage[...] = w16
            pltpu.async_copy(wstage, asel_hbm.at[rankstage], sem2).wait()
            return _

        lax.fori_loop(0, nv, sc_one, 0)

    half = (my_cnt + 1) // 2
    start = c * half
    nc = jnp.where(c == 0, half, my_cnt - half)
    nch = (nc + 15) // 16

    def do_chunk(cc, gstage, rows, sem):
        o = start + cc * 16
        valid = (o + iota) < (start + nc)
        safe0 = plsc.load_gather(idxbuf, [jnp.full((16,), start, i32)])
        idx16 = plsc.load_gather(idxbuf, [jnp.minimum(o + iota,
                                                      my_cnt - 1)])
        gstage[...] = jnp.where(valid, idx16, safe0)
        return pltpu.async_copy(x_hbm.at[gstage], rows, sem)

    def acc_chunk(cc, rows):
        o = start + cc * 16
        valid = (o + iota) < (start + nc)
        v16 = plsc.load_gather(valbuf, [jnp.minimum(o + iota, my_cnt - 1)])
        w16 = jnp.where(valid, jnp.exp(v16 - mval), 0.0) / etot
        wb = [jnp.sum(jnp.where(iota == r, w16, 0.0)) for r in range(16)]

        @pl.loop(0, _GD)
        def _av(v):
            a = acc[v]
            for r in range(16):
                a = a + wb[r] * rows[r, pl.ds(v * 16, 16)]
            acc[v] = a

    @pl.loop(0, nch, step=2)
    def _pair(cc):
        d0 = do_chunk(cc, gstage0, rows0, sem0)

        @pl.when(cc + 1 < nch)
        def _i1():
            do_chunk(cc + 1, gstage1, rows1, sem1)

        d0.wait()
        acc_chunk(cc, rows0)

        @pl.when(cc + 1 < nch)
        def _p1():
            pltpu.make_async_copy(x_hbm.at[gstage1], rows1, sem1).wait()
            acc_chunk(cc + 1, rows1)

    pltpu.sync_copy(acc, mpart_hbm.at[c * _NT + t])


def _sc_select(s1d, meta1d, x):
    mesh = plsc.VectorSubcoreMesh(core_axis_name="c", subcore_axis_name="s")
    cp = pltpu.CompilerParams()
    if "needs_layout_passes" in pltpu.CompilerParams.__dataclass_fields__:
        cp = dataclasses.replace(cp, needs_layout_passes=False)
    f32 = jnp.float32
    i32 = jnp.int32
    kern = functools.partial(
        pl.kernel,
        out_type=(jax.ShapeDtypeStruct((TOPK,), f32),
                  jax.ShapeDtypeStruct((2 * _NT, _GD, 16), f32)),
        mesh=mesh,
        compiler_params=cp,
        scratch_types=[
            pltpu.VMEM((_CH,), f32),
            pltpu.VMEM((_CH + 16,), f32),
            pltpu.VMEM((_CH + 16,), i32),
            pltpu.VMEM((_CH,), f32),
            pltpu.VMEM((1024,), i32),
            pltpu.VMEM((16,), i32),
            pltpu.VMEM((16, 16), i32),
            pltpu.VMEM((_CH,), f32),
            pltpu.VMEM((_CH,), i32),
            pltpu.VMEM((_CH,), f32),
            pltpu.VMEM((16,), i32),
            pltpu.VMEM((16,), f32),
            pltpu.VMEM((16,), i32),
            pltpu.VMEM((16,), i32),
            pltpu.VMEM((16,), f32),
            pltpu.VMEM((16, IN_DIM), f32),
            pltpu.VMEM((16, IN_DIM), f32),
            pltpu.VMEM((_GD, 16), f32),
            pltpu.VMEM_SHARED((_NT, 16), i32),
            pltpu.VMEM_SHARED((_NT, _CH), f32),
            pltpu.SemaphoreType.DMA,
            pltpu.SemaphoreType.DMA,
            pltpu.SemaphoreType.DMA,
        ],
    )(_sc_body)
    return kern(s1d, meta1d, x, jnp.arange(_GD, dtype=i32))


def kernel(x, W1, b1, W2, b2):
    s = _scores(x, W1, b1, W2, b2)
    meta = _thresh(s.reshape(256, 128))
    asel, mpart = _sc_select(s.reshape(N), meta.reshape(1024), x)
    return (jnp.sum(mpart.reshape(2 * _NT, IN_DIM), axis=0), asel[:, None])
